# baseline (speedup 1.0000x reference)
import jax
import jax.numpy as jnp
import numpy as np
from jax import lax
from jax.experimental import pallas as pl
from jax.experimental.pallas import tpu as pltpu
from jax.experimental.pallas import tpu_sc as plsc

_CATS = 26
_D = 32
_BATCH = 16384
_TOTAL = _BATCH * _CATS

_NC, _NS = 2, 16
_NW = _NC * _NS
_PER_W = _TOTAL // _NW

_CHUNK = 1024
_N_CHUNKS = _PER_W // _CHUNK
_IDX_PER_DMA = 128
_DMAS_PER_CHUNK = _CHUNK // _IDX_PER_DMA


def _sc_body(x_hbm, offs_hbm, table_hbm, out_hbm, x_v, offs_v, rows_v, sem):
    wid = lax.axis_index("s") * _NC + lax.axis_index("c")
    base = wid * _PER_W

    pltpu.sync_copy(x_hbm.at[pl.ds(base, _PER_W)], x_v)
    pltpu.sync_copy(offs_hbm, offs_v)

    def add_body(i, _):
        sl = pl.ds(i * 16, 16)
        x_v[sl] = x_v[sl] + offs_v[sl]
        return 0

    lax.fori_loop(0, _PER_W // 16, add_body, 0, unroll=8)

    def chunk_body(k, _):
        start = k * _CHUNK
        for j in range(_DMAS_PER_CHUNK):
            idx_sl = x_v.at[pl.ds(start + j * _IDX_PER_DMA, _IDX_PER_DMA)]
            dst = rows_v.at[pl.ds(j * _IDX_PER_DMA, _IDX_PER_DMA)]
            pltpu.async_copy(table_hbm.at[idx_sl], dst, sem)
        for j in range(_DMAS_PER_CHUNK):
            idx_sl = x_v.at[pl.ds(start + j * _IDX_PER_DMA, _IDX_PER_DMA)]
            dst = rows_v.at[pl.ds(j * _IDX_PER_DMA, _IDX_PER_DMA)]
            pltpu.make_async_copy(table_hbm.at[idx_sl], dst, sem).wait()
        pltpu.sync_copy(rows_v, out_hbm.at[pl.ds(base + start, _CHUNK)])
        return 0

    lax.fori_loop(0, _N_CHUNKS, chunk_body, 0)


_TCOLS = 512
_TGRID = -(-2600000 // _TCOLS)


def _tt_body(in_ref, out_ref):
    out_ref[...] = in_ref[...].T


def _tc_transpose(emb_t):
    return pl.pallas_call(
        _tt_body,
        grid=(_TGRID,),
        in_specs=[pl.BlockSpec((_D, _TCOLS), lambda i: (0, i))],
        out_specs=pl.BlockSpec((_TCOLS, _D), lambda i: (i, 0)),
        out_shape=jax.ShapeDtypeStruct((2600000, _D), jnp.float32),
    )(emb_t)


@jax.jit
def _run(x_flat, offs_tiled, emb_t):
    emb_rows = _tc_transpose(emb_t)
    k = pl.kernel(
        _sc_body,
        out_type=jax.ShapeDtypeStruct((_TOTAL, _D), jnp.float32),
        mesh=plsc.VectorSubcoreMesh(core_axis_name="c", subcore_axis_name="s",
                                    num_cores=_NC, num_subcores=_NS),
        scratch_types=[
            pltpu.VMEM((_PER_W,), jnp.int32),
            pltpu.VMEM((_PER_W,), jnp.int32),
            pltpu.VMEM((_CHUNK, _D), jnp.float32),
            pltpu.SemaphoreType.DMA,
        ],
        compiler_params=pltpu.CompilerParams(use_tc_tiling_on_sc=False),
    )
    return k(x_flat, offs_tiled, emb_rows)


def kernel(x_cat, emb_weight):
    offsets = np.cumsum([0] + [100000] * (_CATS - 1)).astype(np.int32)
    offs_tiled = jnp.asarray(np.tile(offsets, _PER_W // _CATS))
    x_flat = x_cat.reshape(-1)
    out = _run(x_flat, offs_tiled, emb_weight.T)
    return out.reshape(_BATCH, _CATS, _D)


# speedup vs baseline: 2.4863x; 2.4863x over previous
import jax
import jax.numpy as jnp
import numpy as np
from jax import lax
from jax.experimental import pallas as pl
from jax.experimental.pallas import tpu as pltpu
from jax.experimental.pallas import tpu_sc as plsc

_CATS = 26
_D = 32
_BATCH = 16384
_TOTAL = _BATCH * _CATS

_NC, _NS = 2, 16
_NW = _NC * _NS
_PER_W = _TOTAL // _NW

_CHUNK = 1024
_N_CHUNKS = _PER_W // _CHUNK
_IDX_PER_DMA = 128
_DMAS_PER_CHUNK = _CHUNK // _IDX_PER_DMA


def _sc_body(x_hbm, offs_hbm, table_hbm, out_hbm, x_v, offs_v, rows_v, sem):
    wid = lax.axis_index("s") * _NC + lax.axis_index("c")
    base = wid * _PER_W

    pltpu.sync_copy(x_hbm.at[pl.ds(base, _PER_W)], x_v)
    pltpu.sync_copy(offs_hbm, offs_v)

    def add_body(i, _):
        sl = pl.ds(i * 16, 16)
        x_v[sl] = x_v[sl] + offs_v[sl]
        return 0

    lax.fori_loop(0, _PER_W // 16, add_body, 0, unroll=8)

    def chunk_body(k, _):
        start = k * _CHUNK
        for j in range(_DMAS_PER_CHUNK):
            idx_sl = x_v.at[pl.ds(start + j * _IDX_PER_DMA, _IDX_PER_DMA)]
            dst = rows_v.at[pl.ds(j * _IDX_PER_DMA, _IDX_PER_DMA)]
            pltpu.async_copy(table_hbm.at[idx_sl], dst, sem)
        for j in range(_DMAS_PER_CHUNK):
            idx_sl = x_v.at[pl.ds(start + j * _IDX_PER_DMA, _IDX_PER_DMA)]
            dst = rows_v.at[pl.ds(j * _IDX_PER_DMA, _IDX_PER_DMA)]
            pltpu.make_async_copy(table_hbm.at[idx_sl], dst, sem).wait()
        pltpu.sync_copy(rows_v, out_hbm.at[pl.ds(base + start, _CHUNK)])
        return 0

    lax.fori_loop(0, _N_CHUNKS, chunk_body, 0)


_TCOLS = 32768
_TGRID = -(-2600000 // _TCOLS)


def _tt_body(in_ref, out_ref):
    out_ref[...] = in_ref[...].T


def _tc_transpose(emb_t):
    return pl.pallas_call(
        _tt_body,
        grid=(_TGRID,),
        in_specs=[pl.BlockSpec((_D, _TCOLS), lambda i: (0, i))],
        out_specs=pl.BlockSpec((_TCOLS, _D), lambda i: (i, 0)),
        out_shape=jax.ShapeDtypeStruct((2600000, _D), jnp.float32),
    )(emb_t)


@jax.jit
def _run(x_flat, offs_tiled, emb_t):
    emb_rows = _tc_transpose(emb_t)
    k = pl.kernel(
        _sc_body,
        out_type=jax.ShapeDtypeStruct((_TOTAL, _D), jnp.float32),
        mesh=plsc.VectorSubcoreMesh(core_axis_name="c", subcore_axis_name="s",
                                    num_cores=_NC, num_subcores=_NS),
        scratch_types=[
            pltpu.VMEM((_PER_W,), jnp.int32),
            pltpu.VMEM((_PER_W,), jnp.int32),
            pltpu.VMEM((_CHUNK, _D), jnp.float32),
            pltpu.SemaphoreType.DMA,
        ],
        compiler_params=pltpu.CompilerParams(use_tc_tiling_on_sc=False),
    )
    return k(x_flat, offs_tiled, emb_rows)


def kernel(x_cat, emb_weight):
    offsets = np.cumsum([0] + [100000] * (_CATS - 1)).astype(np.int32)
    offs_tiled = jnp.asarray(np.tile(offsets, _PER_W // _CATS))
    x_flat = x_cat.reshape(-1)
    out = _run(x_flat, offs_tiled, emb_weight.T)
    return out.reshape(_BATCH, _CATS, _D)


# speedup vs baseline: 4.4738x; 1.7994x over previous
import jax
import jax.numpy as jnp
import numpy as np
from jax import lax
from jax.experimental import pallas as pl
from jax.experimental.pallas import tpu as pltpu
from jax.experimental.pallas import tpu_sc as plsc

_CATS = 26
_D = 32
_BATCH = 16384
_TOTAL = _BATCH * _CATS

_NC, _NS = 2, 16
_NW = _NC * _NS
_PER_W = _TOTAL // _NW

_CHUNK = 1024
_N_CHUNKS = _PER_W // _CHUNK
_IDX_PER_DMA = 128
_DMAS_PER_CHUNK = _CHUNK // _IDX_PER_DMA


def _sc_body(x_hbm, offs_hbm, table_hbm, out_hbm, x_v, offs_v, rows_v, sem):
    wid = lax.axis_index("s") * _NC + lax.axis_index("c")
    base = wid * _PER_W

    pltpu.sync_copy(x_hbm.at[pl.ds(base, _PER_W)], x_v)
    pltpu.sync_copy(offs_hbm, offs_v)

    def add_body(i, _):
        sl = pl.ds(i * 16, 16)
        r = x_v[sl] + offs_v[sl]
        lo = r & (_TCOLS - 1)
        x_v[sl] = (r - lo) + ((lo & (_TB - 1)) << 2) + (lo >> 13)
        return 0

    lax.fori_loop(0, _PER_W // 16, add_body, 0, unroll=8)

    def chunk_body(k, _):
        start = k * _CHUNK
        for j in range(_DMAS_PER_CHUNK):
            idx_sl = x_v.at[pl.ds(start + j * _IDX_PER_DMA, _IDX_PER_DMA)]
            dst = rows_v.at[pl.ds(j * _IDX_PER_DMA, _IDX_PER_DMA)]
            pltpu.async_copy(table_hbm.at[idx_sl], dst, sem)
        for j in range(_DMAS_PER_CHUNK):
            idx_sl = x_v.at[pl.ds(start + j * _IDX_PER_DMA, _IDX_PER_DMA)]
            dst = rows_v.at[pl.ds(j * _IDX_PER_DMA, _IDX_PER_DMA)]
            pltpu.make_async_copy(table_hbm.at[idx_sl], dst, sem).wait()
        pltpu.sync_copy(rows_v, out_hbm.at[pl.ds(base + start, _CHUNK)])
        return 0

    lax.fori_loop(0, _N_CHUNKS, chunk_body, 0)


_TB = 8192
_TCOLS = 4 * _TB
_TGRID = -(-2600000 // _TCOLS)
_VROWS = _TGRID * _TCOLS


def _tt_body(in_ref, out_ref):
    parts = [in_ref[:, s * _TB:(s + 1) * _TB].T for s in range(4)]
    out_ref[...] = jnp.concatenate(parts, axis=1)


def _tc_transpose(emb_t):
    return pl.pallas_call(
        _tt_body,
        grid=(_TGRID,),
        in_specs=[pl.BlockSpec((_D, _TCOLS), lambda i: (0, i))],
        out_specs=pl.BlockSpec((_TB, 128), lambda i: (i, 0)),
        out_shape=jax.ShapeDtypeStruct((_VROWS // 4, 128), jnp.float32),
    )(emb_t)


@jax.jit
def _run(x_flat, offs_tiled, emb_t):
    emb_rows = _tc_transpose(emb_t).reshape(_VROWS, _D)
    k = pl.kernel(
        _sc_body,
        out_type=jax.ShapeDtypeStruct((_TOTAL, _D), jnp.float32),
        mesh=plsc.VectorSubcoreMesh(core_axis_name="c", subcore_axis_name="s",
                                    num_cores=_NC, num_subcores=_NS),
        scratch_types=[
            pltpu.VMEM((_PER_W,), jnp.int32),
            pltpu.VMEM((_PER_W,), jnp.int32),
            pltpu.VMEM((_CHUNK, _D), jnp.float32),
            pltpu.SemaphoreType.DMA,
        ],
        compiler_params=pltpu.CompilerParams(use_tc_tiling_on_sc=False),
    )
    return k(x_flat, offs_tiled, emb_rows)


def kernel(x_cat, emb_weight):
    offsets = np.cumsum([0] + [100000] * (_CATS - 1)).astype(np.int32)
    offs_tiled = jnp.asarray(np.tile(offsets, _PER_W // _CATS))
    x_flat = x_cat.reshape(-1)
    out = _run(x_flat, offs_tiled, emb_weight.T)
    return out.reshape(_BATCH, _CATS, _D)


# speedup vs baseline: 7.7837x; 1.7398x over previous
import jax
import jax.numpy as jnp
import numpy as np
from jax import lax
from jax.experimental import pallas as pl
from jax.experimental.pallas import tpu as pltpu
from jax.experimental.pallas import tpu_sc as plsc

_CATS = 26
_D = 32
_BATCH = 16384
_TOTAL = _BATCH * _CATS

_NC, _NS = 2, 16
_NW = _NC * _NS
_PER_W = _TOTAL // _NW

_CHUNK = 1024
_N_CHUNKS = _PER_W // _CHUNK
_IDX_PER_DMA = 128
_DMAS_PER_CHUNK = _CHUNK // _IDX_PER_DMA


def _sc_body(x_hbm, offs_hbm, table_hbm, out_hbm, x_v, offs_v, rows_v, sem):
    wid = lax.axis_index("s") * _NC + lax.axis_index("c")
    base = wid * _PER_W

    pltpu.sync_copy(x_hbm.at[pl.ds(base, _PER_W)], x_v)
    pltpu.sync_copy(offs_hbm, offs_v)

    def add_body(i, _):
        sl = pl.ds(i * 16, 16)
        r = x_v[sl] + offs_v[sl]
        lo = r & (_TCOLS - 1)
        x_v[sl] = (r - lo) + ((lo & (_TB - 1)) << 2) + (lo >> 13)
        return 0

    lax.fori_loop(0, _PER_W // 16, add_body, 0, unroll=8)

    def chunk_body(k, _):
        start = k * _CHUNK
        for j in range(_DMAS_PER_CHUNK):
            idx_sl = x_v.at[pl.ds(start + j * _IDX_PER_DMA, _IDX_PER_DMA)]
            dst = rows_v.at[pl.ds(j * _IDX_PER_DMA, _IDX_PER_DMA)]
            pltpu.async_copy(table_hbm.at[idx_sl], dst, sem)
        for j in range(_DMAS_PER_CHUNK):
            idx_sl = x_v.at[pl.ds(start + j * _IDX_PER_DMA, _IDX_PER_DMA)]
            dst = rows_v.at[pl.ds(j * _IDX_PER_DMA, _IDX_PER_DMA)]
            pltpu.make_async_copy(table_hbm.at[idx_sl], dst, sem).wait()
        pltpu.sync_copy(rows_v, out_hbm.at[pl.ds(base + start, _CHUNK)])
        return 0

    lax.fori_loop(0, _N_CHUNKS, chunk_body, 0)


_TB = 8192
_TCOLS = 4 * _TB
_TGRID = -(-2600000 // _TCOLS)
_VROWS = _TGRID * _TCOLS


def _tt_body(in_ref, out_ref):
    stacked = jnp.concatenate(
        [in_ref[:, s * _TB:(s + 1) * _TB] for s in range(4)], axis=0)
    out_ref[...] = stacked.T


def _tc_transpose(emb_t):
    return pl.pallas_call(
        _tt_body,
        grid=(_TGRID,),
        in_specs=[pl.BlockSpec((_D, _TCOLS), lambda i: (0, i))],
        out_specs=pl.BlockSpec((_TB, 128), lambda i: (i, 0)),
        out_shape=jax.ShapeDtypeStruct((_VROWS // 4, 128), jnp.float32),
    )(emb_t)


@jax.jit
def _run(x_flat, offs_tiled, emb_t):
    emb_rows = _tc_transpose(emb_t).reshape(_VROWS, _D)
    k = pl.kernel(
        _sc_body,
        out_type=jax.ShapeDtypeStruct((_TOTAL, _D), jnp.float32),
        mesh=plsc.VectorSubcoreMesh(core_axis_name="c", subcore_axis_name="s",
                                    num_cores=_NC, num_subcores=_NS),
        scratch_types=[
            pltpu.VMEM((_PER_W,), jnp.int32),
            pltpu.VMEM((_PER_W,), jnp.int32),
            pltpu.VMEM((_CHUNK, _D), jnp.float32),
            pltpu.SemaphoreType.DMA,
        ],
        compiler_params=pltpu.CompilerParams(use_tc_tiling_on_sc=False),
    )
    return k(x_flat, offs_tiled, emb_rows)


def kernel(x_cat, emb_weight):
    offsets = np.cumsum([0] + [100000] * (_CATS - 1)).astype(np.int32)
    offs_tiled = jnp.asarray(np.tile(offsets, _PER_W // _CATS))
    x_flat = x_cat.reshape(-1)
    out = _run(x_flat, offs_tiled, emb_weight.T)
    return out.reshape(_BATCH, _CATS, _D)


# speedup vs baseline: 10.2293x; 1.3142x over previous
import jax
import jax.numpy as jnp
import numpy as np
from jax import lax
from jax.experimental import pallas as pl
from jax.experimental.pallas import tpu as pltpu
from jax.experimental.pallas import tpu_sc as plsc

_CATS = 26
_D = 32
_BATCH = 16384
_TOTAL = _BATCH * _CATS

_NC, _NS = 2, 16
_NW = _NC * _NS
_PER_W = _TOTAL // _NW

_CHUNK = 1024
_N_CHUNKS = _PER_W // _CHUNK
_IDX_PER_DMA = 128
_DMAS_PER_CHUNK = _CHUNK // _IDX_PER_DMA


def _sc_body(x_hbm, offs_hbm, scat_hbm, table_hbm, out_hbm,
             x_v, offs_v, scat_v, rows_v, sem, osem):
    wid = lax.axis_index("s") * _NC + lax.axis_index("c")
    base = wid * _PER_W

    pltpu.sync_copy(x_hbm.at[pl.ds(base, _PER_W)], x_v)
    pltpu.sync_copy(offs_hbm, offs_v)
    pltpu.sync_copy(scat_hbm.at[wid], scat_v)

    def add_body(i, _):
        sl = pl.ds(i * 16, 16)
        r = x_v[sl] + offs_v[sl]
        lo = r & (_TCOLS - 1)
        x_v[sl] = (r - lo) + ((lo & (_TB - 1)) << 2) + (lo >> 13)
        return 0

    lax.fori_loop(0, _PER_W // 16, add_body, 0, unroll=8)

    def chunk_body(k, _):
        start = k * _CHUNK
        for j in range(_DMAS_PER_CHUNK):
            idx_sl = x_v.at[pl.ds(start + j * _IDX_PER_DMA, _IDX_PER_DMA)]
            dst = rows_v.at[pl.ds(j * _IDX_PER_DMA, _IDX_PER_DMA)]
            pltpu.async_copy(table_hbm.at[idx_sl], dst, sem)
        for j in range(_DMAS_PER_CHUNK):
            idx_sl = x_v.at[pl.ds(start + j * _IDX_PER_DMA, _IDX_PER_DMA)]
            dst = rows_v.at[pl.ds(j * _IDX_PER_DMA, _IDX_PER_DMA)]
            pltpu.make_async_copy(table_hbm.at[idx_sl], dst, sem).wait()
        for j in range(_DMAS_PER_CHUNK):
            src = rows_v.at[pl.ds(j * _IDX_PER_DMA, _IDX_PER_DMA)]
            idx_sl = scat_v.at[k * _DMAS_PER_CHUNK + j]
            pltpu.async_copy(src, out_hbm.at[idx_sl], osem)
        for j in range(_DMAS_PER_CHUNK):
            src = rows_v.at[pl.ds(j * _IDX_PER_DMA, _IDX_PER_DMA)]
            idx_sl = scat_v.at[k * _DMAS_PER_CHUNK + j]
            pltpu.make_async_copy(src, out_hbm.at[idx_sl], osem).wait()
        return 0

    lax.fori_loop(0, _N_CHUNKS, chunk_body, 0)


_TB = 8192
_TCOLS = 4 * _TB
_TGRID = -(-2600000 // _TCOLS)
_VROWS = _TGRID * _TCOLS


def _tt_body(in_ref, out_ref):
    stacked = jnp.concatenate(
        [in_ref[:, s * _TB:(s + 1) * _TB] for s in range(4)], axis=0)
    out_ref[...] = stacked.T


def _tc_transpose(emb_t):
    return pl.pallas_call(
        _tt_body,
        grid=(_TGRID,),
        in_specs=[pl.BlockSpec((_D, _TCOLS), lambda i: (0, i))],
        out_specs=pl.BlockSpec((_TB, 128), lambda i: (i, 0)),
        out_shape=jax.ShapeDtypeStruct((_VROWS // 4, 128), jnp.float32),
    )(emb_t)


_SLOTS = 32
_OROWS = _BATCH * _SLOTS
_K2BC = 2048


def _k2_body(in_ref, out_ref):
    out_ref[...] = in_ref[...].T


def _tc_finalize(out2):
    v2 = out2.reshape(_OROWS // 4, 128)
    return pl.pallas_call(
        _k2_body,
        grid=(8, _BATCH // _K2BC),
        in_specs=[pl.BlockSpec((_K2BC, 128),
                               lambda g, j: (g * (_BATCH // _K2BC) + j, 0))],
        out_specs=pl.BlockSpec((128, _K2BC), lambda g, j: (g, j)),
        out_shape=jax.ShapeDtypeStruct((1024, _BATCH), jnp.float32),
    )(v2)


@jax.jit
def _run(x_flat, offs_tiled, scat_rows, emb_t):
    emb_rows = _tc_transpose(emb_t).reshape(_VROWS, _D)
    k = pl.kernel(
        _sc_body,
        out_type=jax.ShapeDtypeStruct((_OROWS, _D), jnp.float32),
        mesh=plsc.VectorSubcoreMesh(core_axis_name="c", subcore_axis_name="s",
                                    num_cores=_NC, num_subcores=_NS),
        scratch_types=[
            pltpu.VMEM((_PER_W,), jnp.int32),
            pltpu.VMEM((_PER_W,), jnp.int32),
            pltpu.VMEM((_PER_W // 128, 128), jnp.int32),
            pltpu.VMEM((_CHUNK, _D), jnp.float32),
            pltpu.SemaphoreType.DMA,
            pltpu.SemaphoreType.DMA,
        ],
        compiler_params=pltpu.CompilerParams(use_tc_tiling_on_sc=False),
    )
    out2 = k(x_flat, offs_tiled, scat_rows, emb_rows)
    res = _tc_finalize(out2)
    return res.reshape(_SLOTS, _D, _BATCH)[:_CATS].transpose(2, 0, 1)


def kernel(x_cat, emb_weight):
    offsets = np.cumsum([0] + [100000] * (_CATS - 1)).astype(np.int32)
    offs_tiled = jnp.asarray(np.tile(offsets, _PER_W // _CATS))
    n = np.arange(_TOTAL, dtype=np.int64)
    b, c = n // _CATS, n % _CATS
    scat = ((c // 4) * (4 * _BATCH) + 4 * b + c % 4).astype(np.int32)
    scat_rows = jnp.asarray(scat.reshape(_NW, _PER_W // 128, 128))
    x_flat = x_cat.reshape(-1)
    return _run(x_flat, offs_tiled, scat_rows, emb_weight.T)


# speedup vs baseline: 12.0587x; 1.1788x over previous
import jax
import jax.numpy as jnp
import numpy as np
from jax import lax
from jax.experimental import pallas as pl
from jax.experimental.pallas import tpu as pltpu
from jax.experimental.pallas import tpu_sc as plsc

_CATS = 26
_D = 32
_BATCH = 16384
_TOTAL = _BATCH * _CATS

_NC, _NS = 2, 16
_NW = _NC * _NS
_PER_W = _TOTAL // _NW

_CHUNK = 1024
_N_CHUNKS = _PER_W // _CHUNK
_IDX_PER_DMA = 128
_DMAS_PER_CHUNK = _CHUNK // _IDX_PER_DMA


def _sc_body(x_hbm, offs_hbm, scat_hbm, table_hbm, out_hbm,
             x_v, offs_v, scat_v, rows_v, sem, osem):
    wid = lax.axis_index("s") * _NC + lax.axis_index("c")
    base = wid * _PER_W

    pltpu.sync_copy(x_hbm.at[pl.ds(base, _PER_W)], x_v)
    pltpu.sync_copy(offs_hbm, offs_v)
    pltpu.sync_copy(scat_hbm.at[wid], scat_v)

    def add_body(i, _):
        sl = pl.ds(i * 16, 16)
        r = x_v[sl] + offs_v[sl]
        lo = r & (_TCOLS - 1)
        x_v[sl] = (r - lo) + ((lo & (_TB - 1)) << 2) + (lo >> 13)
        return 0

    lax.fori_loop(0, _PER_W // 16, add_body, 0, unroll=8)

    def chunk_body(k, _):
        start = k * _CHUNK
        for j in range(_DMAS_PER_CHUNK):
            idx_sl = x_v.at[pl.ds(start + j * _IDX_PER_DMA, _IDX_PER_DMA)]
            dst = rows_v.at[pl.ds(j * _IDX_PER_DMA, _IDX_PER_DMA)]
            pltpu.async_copy(table_hbm.at[idx_sl], dst, sem)
        for j in range(_DMAS_PER_CHUNK):
            idx_sl = x_v.at[pl.ds(start + j * _IDX_PER_DMA, _IDX_PER_DMA)]
            dst = rows_v.at[pl.ds(j * _IDX_PER_DMA, _IDX_PER_DMA)]
            pltpu.make_async_copy(table_hbm.at[idx_sl], dst, sem).wait()
        for j in range(_DMAS_PER_CHUNK):
            src = rows_v.at[pl.ds(j * _IDX_PER_DMA, _IDX_PER_DMA)]
            idx_sl = scat_v.at[k * _DMAS_PER_CHUNK + j]
            pltpu.async_copy(src, out_hbm.at[idx_sl], osem)
        for j in range(_DMAS_PER_CHUNK):
            src = rows_v.at[pl.ds(j * _IDX_PER_DMA, _IDX_PER_DMA)]
            idx_sl = scat_v.at[k * _DMAS_PER_CHUNK + j]
            pltpu.make_async_copy(src, out_hbm.at[idx_sl], osem).wait()
        return 0

    lax.fori_loop(0, _N_CHUNKS, chunk_body, 0)


_TB = 8192
_TCOLS = 4 * _TB
_TGRID = -(-2600000 // _TCOLS)
_VROWS = _TGRID * _TCOLS


def _tt_body(in_ref, out_ref):
    stacked = jnp.concatenate(
        [in_ref[:, s * _TB:(s + 1) * _TB] for s in range(4)], axis=0)
    out_ref[...] = stacked.T


def _tc_transpose(emb_t):
    return pl.pallas_call(
        _tt_body,
        grid=(_TGRID,),
        in_specs=[pl.BlockSpec((_D, _TCOLS), lambda i: (0, i))],
        out_specs=pl.BlockSpec((_TB, 128), lambda i: (i, 0)),
        out_shape=jax.ShapeDtypeStruct((_VROWS // 4, 128), jnp.float32),
    )(emb_t)


_SLOTS = 32
_OROWS = _BATCH * _SLOTS
_K2BC = 4096


def _k2_body(in_ref, out_ref):
    out_ref[...] = in_ref[...].T


def _tc_finalize(out2):
    v2 = out2.reshape(_OROWS // 4, 128)
    return pl.pallas_call(
        _k2_body,
        grid=(7, _BATCH // _K2BC),
        in_specs=[pl.BlockSpec((_K2BC, 128),
                               lambda g, j: (g * (_BATCH // _K2BC) + j, 0))],
        out_specs=pl.BlockSpec((128, _K2BC), lambda g, j: (g, j)),
        out_shape=jax.ShapeDtypeStruct((832, _BATCH), jnp.float32),
    )(v2)


@jax.jit
def _run(x_flat, offs_tiled, scat_rows, emb_t):
    emb_rows = _tc_transpose(emb_t).reshape(_VROWS, _D)
    k = pl.kernel(
        _sc_body,
        out_type=jax.ShapeDtypeStruct((_OROWS, _D), jnp.float32),
        mesh=plsc.VectorSubcoreMesh(core_axis_name="c", subcore_axis_name="s",
                                    num_cores=_NC, num_subcores=_NS),
        scratch_types=[
            pltpu.VMEM((_PER_W,), jnp.int32),
            pltpu.VMEM((_PER_W,), jnp.int32),
            pltpu.VMEM((_PER_W // 128, 128), jnp.int32),
            pltpu.VMEM((_CHUNK, _D), jnp.float32),
            pltpu.SemaphoreType.DMA,
            pltpu.SemaphoreType.DMA,
        ],
        compiler_params=pltpu.CompilerParams(use_tc_tiling_on_sc=False),
    )
    out2 = k(x_flat, offs_tiled, scat_rows, emb_rows)
    res = _tc_finalize(out2)
    return res.reshape(_CATS, _D, _BATCH).transpose(2, 0, 1)


def kernel(x_cat, emb_weight):
    offsets = np.cumsum([0] + [100000] * (_CATS - 1)).astype(np.int32)
    offs_tiled = jnp.asarray(np.tile(offsets, _PER_W // _CATS))
    n = np.arange(_TOTAL, dtype=np.int64)
    b, c = n // _CATS, n % _CATS
    scat = ((c // 4) * (4 * _BATCH) + 4 * b + c % 4).astype(np.int32)
    scat_rows = jnp.asarray(scat.reshape(_NW, _PER_W // 128, 128))
    x_flat = x_cat.reshape(-1)
    return _run(x_flat, offs_tiled, scat_rows, emb_weight.T)


# speedup vs baseline: 12.4176x; 1.0298x over previous
import jax
import jax.numpy as jnp
import numpy as np
from jax import lax
from jax.experimental import pallas as pl
from jax.experimental.pallas import tpu as pltpu
from jax.experimental.pallas import tpu_sc as plsc

_CATS = 26
_D = 32
_BATCH = 16384
_TOTAL = _BATCH * _CATS

_NC, _NS = 2, 16
_NW = _NC * _NS
_PER_W = _TOTAL // _NW

_CHUNK = 1024
_N_CHUNKS = _PER_W // _CHUNK
_IDX_PER_DMA = 128
_DMAS_PER_CHUNK = _CHUNK // _IDX_PER_DMA


def _sc_body(x_hbm, offs_hbm, scat_hbm, table_hbm, out_hbm,
             x_v, offs_v, scat_v, rows_v, sem, osem):
    wid = lax.axis_index("s") * _NC + lax.axis_index("c")
    base = wid * _PER_W

    pltpu.sync_copy(x_hbm.at[pl.ds(base, _PER_W)], x_v)
    pltpu.sync_copy(offs_hbm, offs_v)
    pltpu.sync_copy(scat_hbm.at[wid], scat_v)

    def add_body(i, _):
        sl = pl.ds(i * 16, 16)
        r = x_v[sl] + offs_v[sl]
        lo = r & (_TCOLS - 1)
        x_v[sl] = (r - lo) + ((lo & (_TB - 1)) << 2) + (lo >> _TBLOG)
        return 0

    lax.fori_loop(0, _PER_W // 16, add_body, 0, unroll=8)

    def chunk_body(k, _):
        start = k * _CHUNK
        for j in range(_DMAS_PER_CHUNK):
            idx_sl = x_v.at[pl.ds(start + j * _IDX_PER_DMA, _IDX_PER_DMA)]
            dst = rows_v.at[pl.ds(j * _IDX_PER_DMA, _IDX_PER_DMA)]
            pltpu.async_copy(table_hbm.at[idx_sl], dst, sem)
        for j in range(_DMAS_PER_CHUNK):
            idx_sl = x_v.at[pl.ds(start + j * _IDX_PER_DMA, _IDX_PER_DMA)]
            dst = rows_v.at[pl.ds(j * _IDX_PER_DMA, _IDX_PER_DMA)]
            pltpu.make_async_copy(table_hbm.at[idx_sl], dst, sem).wait()
        for j in range(_DMAS_PER_CHUNK):
            src = rows_v.at[pl.ds(j * _IDX_PER_DMA, _IDX_PER_DMA)]
            idx_sl = scat_v.at[k * _DMAS_PER_CHUNK + j]
            pltpu.async_copy(src, out_hbm.at[idx_sl], osem)
        for j in range(_DMAS_PER_CHUNK):
            src = rows_v.at[pl.ds(j * _IDX_PER_DMA, _IDX_PER_DMA)]
            idx_sl = scat_v.at[k * _DMAS_PER_CHUNK + j]
            pltpu.make_async_copy(src, out_hbm.at[idx_sl], osem).wait()
        return 0

    lax.fori_loop(0, _N_CHUNKS, chunk_body, 0)


_TB = 16384
_TBLOG = 14
_TCOLS = 4 * _TB
_TGRID = -(-2600000 // _TCOLS)
_VROWS = _TGRID * _TCOLS


def _tt_body(in_ref, out_ref):
    stacked = jnp.concatenate(
        [in_ref[:, s * _TB:(s + 1) * _TB] for s in range(4)], axis=0)
    out_ref[...] = stacked.T


def _tc_transpose(emb_t):
    return pl.pallas_call(
        _tt_body,
        grid=(_TGRID,),
        in_specs=[pl.BlockSpec((_D, _TCOLS), lambda i: (0, i))],
        out_specs=pl.BlockSpec((_TB, 128), lambda i: (i, 0)),
        out_shape=jax.ShapeDtypeStruct((_VROWS // 4, 128), jnp.float32),
    )(emb_t)


_SLOTS = 32
_OROWS = _BATCH * _SLOTS
_K2BC = 8192


def _k2_body(in_ref, out_ref):
    out_ref[...] = in_ref[...].T


def _tc_finalize(out2):
    v2 = out2.reshape(_OROWS // 4, 128)
    return pl.pallas_call(
        _k2_body,
        grid=(7, _BATCH // _K2BC),
        in_specs=[pl.BlockSpec((_K2BC, 128),
                               lambda g, j: (g * (_BATCH // _K2BC) + j, 0))],
        out_specs=pl.BlockSpec((128, _K2BC), lambda g, j: (g, j)),
        out_shape=jax.ShapeDtypeStruct((832, _BATCH), jnp.float32),
    )(v2)


@jax.jit
def _run(x_flat, offs_tiled, scat_rows, emb_t):
    emb_rows = _tc_transpose(emb_t).reshape(_VROWS, _D)
    k = pl.kernel(
        _sc_body,
        out_type=jax.ShapeDtypeStruct((_OROWS, _D), jnp.float32),
        mesh=plsc.VectorSubcoreMesh(core_axis_name="c", subcore_axis_name="s",
                                    num_cores=_NC, num_subcores=_NS),
        scratch_types=[
            pltpu.VMEM((_PER_W,), jnp.int32),
            pltpu.VMEM((_PER_W,), jnp.int32),
            pltpu.VMEM((_PER_W // 128, 128), jnp.int32),
            pltpu.VMEM((_CHUNK, _D), jnp.float32),
            pltpu.SemaphoreType.DMA,
            pltpu.SemaphoreType.DMA,
        ],
        compiler_params=pltpu.CompilerParams(use_tc_tiling_on_sc=False),
    )
    out2 = k(x_flat, offs_tiled, scat_rows, emb_rows)
    res = _tc_finalize(out2)
    return res.reshape(_CATS, _D, _BATCH).transpose(2, 0, 1)


def kernel(x_cat, emb_weight):
    offsets = np.cumsum([0] + [100000] * (_CATS - 1)).astype(np.int32)
    offs_tiled = jnp.asarray(np.tile(offsets, _PER_W // _CATS))
    n = np.arange(_TOTAL, dtype=np.int64)
    b, c = n // _CATS, n % _CATS
    scat = ((c // 4) * (4 * _BATCH) + 4 * b + c % 4).astype(np.int32)
    scat_rows = jnp.asarray(scat.reshape(_NW, _PER_W // 128, 128))
    x_flat = x_cat.reshape(-1)
    return _run(x_flat, offs_tiled, scat_rows, emb_weight.T)


# speedup vs baseline: 12.6870x; 1.0217x over previous
import jax
import jax.numpy as jnp
import numpy as np
from jax import lax
from jax.experimental import pallas as pl
from jax.experimental.pallas import tpu as pltpu
from jax.experimental.pallas import tpu_sc as plsc

_CATS = 26
_D = 32
_BATCH = 16384
_TOTAL = _BATCH * _CATS

_NC, _NS = 2, 16
_NW = _NC * _NS
_PER_W = _TOTAL // _NW

_CHUNK = 1024
_N_CHUNKS = _PER_W // _CHUNK
_IDX_PER_DMA = 128
_DMAS_PER_CHUNK = _CHUNK // _IDX_PER_DMA


def _sc_body(x_hbm, offs_hbm, scat_hbm, table_hbm, out_hbm,
             x_v, offs_v, scat_v, rows_a, rows_b, sem, osem):
    wid = lax.axis_index("s") * _NC + lax.axis_index("c")
    base = wid * _PER_W

    pltpu.sync_copy(x_hbm.at[pl.ds(base, _PER_W)], x_v)
    pltpu.sync_copy(offs_hbm, offs_v)
    pltpu.sync_copy(scat_hbm.at[wid], scat_v)

    def add_body(i, _):
        sl = pl.ds(i * 16, 16)
        r = x_v[sl] + offs_v[sl]
        lo = r & (_TCOLS - 1)
        x_v[sl] = (r - lo) + ((lo & (_TB - 1)) << 2) + (lo >> _TBLOG)
        return 0

    lax.fori_loop(0, _PER_W // 16, add_body, 0, unroll=8)

    def gather_piece(k, buf, j, wait):
        start = k * _CHUNK
        idx_sl = x_v.at[pl.ds(start + j * _IDX_PER_DMA, _IDX_PER_DMA)]
        dst = buf.at[pl.ds(j * _IDX_PER_DMA, _IDX_PER_DMA)]
        cp = pltpu.make_async_copy(table_hbm.at[idx_sl], dst, sem)
        cp.wait() if wait else cp.start()

    def scatter_piece(k, buf, j, wait):
        src = buf.at[pl.ds(j * _IDX_PER_DMA, _IDX_PER_DMA)]
        idx_sl = scat_v.at[k * _DMAS_PER_CHUNK + j]
        cp = pltpu.make_async_copy(src, out_hbm.at[idx_sl], osem)
        cp.wait() if wait else cp.start()

    bufs = (rows_a, rows_b)
    for j in range(_DMAS_PER_CHUNK):
        gather_piece(0, bufs[0], j, False)
    for k in range(_N_CHUNKS):
        buf, nxt = bufs[k % 2], bufs[(k + 1) % 2]
        for j in range(_DMAS_PER_CHUNK):
            gather_piece(k, buf, j, True)
        if k >= 1:
            for j in range(_DMAS_PER_CHUNK):
                scatter_piece(k - 1, nxt, j, True)
        if k + 1 < _N_CHUNKS:
            for j in range(_DMAS_PER_CHUNK):
                gather_piece(k + 1, nxt, j, False)
        for j in range(_DMAS_PER_CHUNK):
            scatter_piece(k, buf, j, False)
    for j in range(_DMAS_PER_CHUNK):
        scatter_piece(_N_CHUNKS - 1, bufs[(_N_CHUNKS - 1) % 2], j, True)


_TB = 16384
_TBLOG = 14
_TCOLS = 4 * _TB
_TGRID = -(-2600000 // _TCOLS)
_VROWS = _TGRID * _TCOLS


def _tt_body(in_ref, out_ref):
    stacked = jnp.concatenate(
        [in_ref[:, s * _TB:(s + 1) * _TB] for s in range(4)], axis=0)
    out_ref[...] = stacked.T


def _tc_transpose(emb_t):
    return pl.pallas_call(
        _tt_body,
        grid=(_TGRID,),
        in_specs=[pl.BlockSpec((_D, _TCOLS), lambda i: (0, i))],
        out_specs=pl.BlockSpec((_TB, 128), lambda i: (i, 0)),
        out_shape=jax.ShapeDtypeStruct((_VROWS // 4, 128), jnp.float32),
    )(emb_t)


_SLOTS = 32
_OROWS = _BATCH * _SLOTS
_K2BC = 8192


def _k2_body(in_ref, out_ref):
    out_ref[...] = in_ref[...].T


def _tc_finalize(out2):
    v2 = out2.reshape(_OROWS // 4, 128)
    return pl.pallas_call(
        _k2_body,
        grid=(7, _BATCH // _K2BC),
        in_specs=[pl.BlockSpec((_K2BC, 128),
                               lambda g, j: (g * (_BATCH // _K2BC) + j, 0))],
        out_specs=pl.BlockSpec((128, _K2BC), lambda g, j: (g, j)),
        out_shape=jax.ShapeDtypeStruct((832, _BATCH), jnp.float32),
    )(v2)


@jax.jit
def _run(x_flat, offs_tiled, scat_rows, emb_t):
    emb_rows = _tc_transpose(emb_t).reshape(_VROWS, _D)
    k = pl.kernel(
        _sc_body,
        out_type=jax.ShapeDtypeStruct((_OROWS, _D), jnp.float32),
        mesh=plsc.VectorSubcoreMesh(core_axis_name="c", subcore_axis_name="s",
                                    num_cores=_NC, num_subcores=_NS),
        scratch_types=[
            pltpu.VMEM((_PER_W,), jnp.int32),
            pltpu.VMEM((_PER_W,), jnp.int32),
            pltpu.VMEM((_PER_W // 128, 128), jnp.int32),
            pltpu.VMEM((_CHUNK, _D), jnp.float32),
            pltpu.VMEM((_CHUNK, _D), jnp.float32),
            pltpu.SemaphoreType.DMA,
            pltpu.SemaphoreType.DMA,
        ],
        compiler_params=pltpu.CompilerParams(use_tc_tiling_on_sc=False),
    )
    out2 = k(x_flat, offs_tiled, scat_rows, emb_rows)
    res = _tc_finalize(out2)
    return res.reshape(_CATS, _D, _BATCH).transpose(2, 0, 1)


def kernel(x_cat, emb_weight):
    offsets = np.cumsum([0] + [100000] * (_CATS - 1)).astype(np.int32)
    offs_tiled = jnp.asarray(np.tile(offsets, _PER_W // _CATS))
    n = np.arange(_TOTAL, dtype=np.int64)
    b, c = n // _CATS, n % _CATS
    scat = ((c // 4) * (4 * _BATCH) + 4 * b + c % 4).astype(np.int32)
    scat_rows = jnp.asarray(scat.reshape(_NW, _PER_W // 128, 128))
    x_flat = x_cat.reshape(-1)
    return _run(x_flat, offs_tiled, scat_rows, emb_weight.T)
